# Initial kernel scaffold; baseline (speedup 1.0000x reference)
import jax
import jax.numpy as jnp
from jax.experimental import pallas as pl


def kernel(pred_cls, pred_box, anchors, mask, tgt_boxes, tgt_labels):
    raise NotImplementedError("write your pallas kernel here")


# speedup vs baseline: 1.7369x; 1.7369x over previous
import jax
import jax.numpy as jnp
from jax import lax
from jax.experimental import pallas as pl
from jax.experimental.pallas import tpu as pltpu

_NUM_CLASSES = 80
_ALPHA, _GAMMA = 0.25, 2.0
_TOPK = 4
_IGNORE_THRESH, _IOU_THRESH = 0.7, 0.15
_BIG = 3.0e38


def _match_kernel(pred_t_ref, anc_t_ref, tgtn_ref, out_ref):
    m = pred_t_ref.shape[2]
    g = tgtn_ref.shape[1]
    x0 = pred_t_ref[0, 0:1, :]
    y0 = pred_t_ref[0, 1:2, :]
    x1 = pred_t_ref[0, 2:3, :]
    y1 = pred_t_ref[0, 3:4, :]
    pcx = (x0 + x1) * 0.5
    pcy = (y0 + y1) * 0.5
    pw = x1 - x0
    ph = y1 - y0

    tb = tgtn_ref[0]
    tx0 = tb[:, 0:1]
    ty0 = tb[:, 1:2]
    tx1 = tb[:, 2:3]
    ty1 = tb[:, 3:4]
    tcx = (tx0 + tx1) * 0.5
    tcy = (ty0 + ty1) * 0.5
    tw = tx1 - tx0
    th = ty1 - ty0

    col = lax.broadcasted_iota(jnp.int32, (1, m), 1).astype(jnp.float32)

    def top4(c):
        cols = []
        for _ in range(_TOPK):
            v = jnp.min(c, axis=1, keepdims=True)
            idx = jnp.min(jnp.where(c <= v, col, _BIG), axis=1, keepdims=True)
            cols.append(idx)
            c = jnp.where(col == idx, _BIG, c)
        return cols

    cost_p = (jnp.abs(pcx - tcx) + jnp.abs(pcy - tcy)
              + jnp.abs(pw - tw) + jnp.abs(ph - th))
    acx = anc_t_ref[0:1, :]
    acy = anc_t_ref[1:2, :]
    aw = anc_t_ref[2:3, :]
    ah = anc_t_ref[3:4, :]
    cost_a = (jnp.abs(acx - tcx) + jnp.abs(acy - tcy)
              + jnp.abs(aw - tw) + jnp.abs(ah - th))
    cols = top4(cost_p) + top4(cost_a)
    out_ref[0] = jnp.concatenate(cols, axis=1).astype(jnp.int32)


def _loss_kernel(pc_ref, pb_ref, anc_ref, tgt_t_ref, tgtj_ref, labj_ref,
                 srcr_ref, srcc_ref, mskf_ref, out_ref, acc_ref):
    b = pl.program_id(0)
    t = pl.program_id(1)
    nb = pl.num_programs(0)
    nt = pl.num_programs(1)
    tile = pb_ref.shape[1]
    c_dim = pc_ref.shape[2]

    @pl.when(jnp.logical_and(b == 0, t == 0))
    def _():
        acc_ref[0] = 0.0
        acc_ref[1] = 0.0
        acc_ref[2] = 0.0

    base = (t * tile).astype(jnp.float32)
    a_col = lax.broadcasted_iota(jnp.int32, (tile, 1), 0).astype(jnp.float32) + base
    src_r = srcr_ref[0].astype(jnp.float32)
    src_c = srcc_ref[0].astype(jnp.float32)
    j_n = srcc_ref.shape[1]

    mask_aj = (a_col == src_r).astype(jnp.float32)
    in_tile = jnp.logical_and(src_c >= base, src_c < base + tile).astype(jnp.float32)

    vals8 = jnp.concatenate([anc_ref[...], pb_ref[0]], axis=1)
    gath = lax.dot_general(mask_aj, vals8, (((0,), (0,)), ((), ())),
                           preferred_element_type=jnp.float32)
    acx = gath[:, 0:1]
    acy = gath[:, 1:2]
    aw = gath[:, 2:3]
    ah = gath[:, 3:4]
    ax0 = acx - 0.5 * aw
    ay0 = acy - 0.5 * ah
    ax1 = acx + 0.5 * aw
    ay1 = acy + 0.5 * ah
    px0 = gath[:, 4:5]
    py0 = gath[:, 5:6]
    px1 = gath[:, 6:7]
    py1 = gath[:, 7:8]

    tj = tgtj_ref[0]
    tx0 = tj[:, 0:1]
    ty0 = tj[:, 1:2]
    tx1 = tj[:, 2:3]
    ty1 = tj[:, 3:4]
    area_t = (tx1 - tx0) * (ty1 - ty0)

    area_a = (ax1 - ax0) * (ay1 - ay0)
    iw = jnp.maximum(jnp.minimum(ax1, tx1) - jnp.maximum(ax0, tx0), 0.0)
    ih = jnp.maximum(jnp.minimum(ay1, ty1) - jnp.maximum(ay0, ty0), 0.0)
    inter = iw * ih
    union = area_a + area_t - inter
    pos_iou = inter / jnp.maximum(union, 1e-8)

    keep = (pos_iou >= _IOU_THRESH).astype(jnp.float32)
    tgt_cls_o = jnp.where(pos_iou < _IOU_THRESH, -1.0, labj_ref[0])

    j_col = lax.broadcasted_iota(jnp.int32, (j_n, 1), 0).astype(jnp.float32)
    j_row = lax.broadcasted_iota(jnp.int32, (1, j_n), 1).astype(jnp.float32)
    later = jnp.max(
        jnp.where(jnp.logical_and(src_c == src_r, j_row > j_col), 1.0, 0.0),
        axis=1, keepdims=True)
    is_last = 1.0 - later

    enc2 = jnp.concatenate([tgt_cls_o * is_last, is_last], axis=1)
    agg = lax.dot_general(mask_aj, enc2, (((1,), (0,)), ((), ())),
                          preferred_element_type=jnp.float32)
    enc_sum = agg[:, 0:1]
    matched = agg[:, 1:2] > 0.5

    pb = pb_ref[0]
    qx0 = pb[:, 0:1]
    qy0 = pb[:, 1:2]
    qx1 = pb[:, 2:3]
    qy1 = pb[:, 3:4]
    gx0 = tgt_t_ref[0, 0:1, :]
    gy0 = tgt_t_ref[0, 1:2, :]
    gx1 = tgt_t_ref[0, 2:3, :]
    gy1 = tgt_t_ref[0, 3:4, :]
    area_q = (qx1 - qx0) * (qy1 - qy0)
    area_g = (gx1 - gx0) * (gy1 - gy0)
    iw2 = jnp.maximum(jnp.minimum(qx1, gx1) - jnp.maximum(qx0, gx0), 0.0)
    ih2 = jnp.maximum(jnp.minimum(qy1, gy1) - jnp.maximum(qy0, gy0), 0.0)
    inter2 = iw2 * ih2
    union2 = area_q + area_g - inter2
    iou2 = inter2 / jnp.maximum(union2, 1e-8)
    iou_max = jnp.max(iou2, axis=1, keepdims=True)

    gt_cls = jnp.where(matched, enc_sum,
                       jnp.where(iou_max > _IGNORE_THRESH, -1.0,
                                 float(_NUM_CLASSES)))
    valid_f = (gt_cls >= 0.0).astype(jnp.float32) * (1.0 - mskf_ref[0])
    fg_f = jnp.logical_and(gt_cls >= 0.0,
                           gt_cls < _NUM_CLASSES - 0.5).astype(jnp.float32)
    fg_cls = gt_cls * fg_f

    cls_iota = lax.broadcasted_iota(jnp.int32, (tile, c_dim), 1).astype(jnp.float32)
    t_mat = (cls_iota == fg_cls).astype(jnp.float32) * fg_f
    x = pc_ref[0]
    p = jax.nn.sigmoid(x)
    ce = jnp.maximum(x, 0.0) - x * t_mat + jnp.log1p(jnp.exp(-jnp.abs(x)))
    p_t = p * t_mat + (1.0 - p) * (1.0 - t_mat)
    one_m = 1.0 - p_t
    focal = ce * (one_m * one_m)
    alpha_t = _ALPHA * t_mat + (1.0 - _ALPHA) * (1.0 - t_mat)
    cls_part = jnp.sum(alpha_t * focal * valid_f)

    area_p2 = (px1 - px0) * (py1 - py0)
    iw3 = jnp.maximum(jnp.minimum(px1, tx1) - jnp.maximum(px0, tx0), 0.0)
    ih3 = jnp.maximum(jnp.minimum(py1, ty1) - jnp.maximum(py0, ty0), 0.0)
    inter3 = iw3 * ih3
    union3 = area_p2 + area_t - inter3
    iou3 = inter3 / jnp.maximum(union3, 1e-8)
    ew = jnp.maximum(jnp.maximum(px1, tx1) - jnp.minimum(px0, tx0), 0.0)
    eh = jnp.maximum(jnp.maximum(py1, ty1) - jnp.minimum(py0, ty0), 0.0)
    area_e = ew * eh
    gi = iou3 - (area_e - union3) / jnp.maximum(area_e, 1e-8)
    reg_part = jnp.sum(in_tile * keep * (1.0 - gi))

    acc_ref[0] = acc_ref[0] + cls_part
    acc_ref[1] = acc_ref[1] + reg_part
    acc_ref[2] = acc_ref[2] + jnp.sum(fg_f)

    @pl.when(jnp.logical_and(b == nb - 1, t == nt - 1))
    def _():
        num_fg = jnp.maximum(acc_ref[2], 1.0)
        lc = acc_ref[0] / num_fg
        lr = acc_ref[1] / num_fg
        out_ref[0] = lc
        out_ref[1] = lr
        out_ref[2] = lc + lr


def kernel(pred_cls, pred_box, anchors, mask, tgt_boxes, tgt_labels):
    bn, m, c = pred_cls.shape
    g = tgt_boxes.shape[1]
    j_n = 2 * _TOPK * g

    pred_box_t = pred_box.transpose(0, 2, 1)
    anchors_t = anchors.T
    tgt_t = tgt_boxes.transpose(0, 2, 1)

    matches = pl.pallas_call(
        _match_kernel,
        grid=(bn,),
        in_specs=[
            pl.BlockSpec((1, 4, m), lambda b: (b, 0, 0)),
            pl.BlockSpec((4, m), lambda b: (0, 0)),
            pl.BlockSpec((1, g, 4), lambda b: (b, 0, 0)),
        ],
        out_specs=pl.BlockSpec((1, g, 2 * _TOPK), lambda b: (b, 0, 0)),
        out_shape=jax.ShapeDtypeStruct((bn, g, 2 * _TOPK), jnp.int32),
    )(pred_box_t, anchors_t, tgt_boxes)

    m_t = matches.transpose(0, 2, 1)
    idx_p = m_t[:, :_TOPK, :]
    idx_a = m_t[:, _TOPK:, :]
    src = jnp.stack([idx_p, idx_a], axis=2).reshape(bn, j_n)
    src_row = src.reshape(bn, 1, j_n)
    src_col = src.reshape(bn, j_n, 1)
    tgt_j = jnp.tile(tgt_boxes, (1, 2 * _TOPK, 1))
    lab_j = jnp.tile(tgt_labels.astype(jnp.float32),
                     (1, 2 * _TOPK)).reshape(bn, j_n, 1)
    mask_f = mask.astype(jnp.float32).reshape(bn, m, 1)

    tile = 2048
    t_n = m // tile
    out = pl.pallas_call(
        _loss_kernel,
        grid=(bn, t_n),
        in_specs=[
            pl.BlockSpec((1, tile, c), lambda b, t: (b, t, 0)),
            pl.BlockSpec((1, tile, 4), lambda b, t: (b, t, 0)),
            pl.BlockSpec((tile, 4), lambda b, t: (t, 0)),
            pl.BlockSpec((1, 4, g), lambda b, t: (b, 0, 0)),
            pl.BlockSpec((1, j_n, 4), lambda b, t: (b, 0, 0)),
            pl.BlockSpec((1, j_n, 1), lambda b, t: (b, 0, 0)),
            pl.BlockSpec((1, 1, j_n), lambda b, t: (b, 0, 0)),
            pl.BlockSpec((1, j_n, 1), lambda b, t: (b, 0, 0)),
            pl.BlockSpec((1, tile, 1), lambda b, t: (b, t, 0)),
        ],
        out_specs=pl.BlockSpec(memory_space=pltpu.SMEM),
        out_shape=jax.ShapeDtypeStruct((3,), jnp.float32),
        scratch_shapes=[pltpu.SMEM((3,), jnp.float32)],
    )(pred_cls, pred_box, anchors, tgt_t, tgt_j, lab_j, src_row, src_col,
      mask_f)

    return out[0], out[1], out[2]


# speedup vs baseline: 2.0852x; 1.2005x over previous
import jax
import jax.numpy as jnp
from jax import lax
from jax.experimental import pallas as pl
from jax.experimental.pallas import tpu as pltpu

_NUM_CLASSES = 80
_ALPHA, _GAMMA = 0.25, 2.0
_TOPK = 4
_IGNORE_THRESH, _IOU_THRESH = 0.7, 0.15
_BIG = 3.0e38


def _match_kernel(pred_t_ref, anc_t_ref, tgtn_ref, out_ref, last_ref):
    m = pred_t_ref.shape[2]
    g = tgtn_ref.shape[1]
    j_n = 2 * _TOPK * g
    x0 = pred_t_ref[0, 0:1, :]
    y0 = pred_t_ref[0, 1:2, :]
    x1 = pred_t_ref[0, 2:3, :]
    y1 = pred_t_ref[0, 3:4, :]
    pcx = (x0 + x1) * 0.5
    pcy = (y0 + y1) * 0.5
    pw = x1 - x0
    ph = y1 - y0

    tb = tgtn_ref[0]
    tx0 = tb[:, 0:1]
    ty0 = tb[:, 1:2]
    tx1 = tb[:, 2:3]
    ty1 = tb[:, 3:4]
    tcx = (tx0 + tx1) * 0.5
    tcy = (ty0 + ty1) * 0.5
    tw = tx1 - tx0
    th = ty1 - ty0

    col = lax.broadcasted_iota(jnp.int32, (1, m), 1).astype(jnp.float32)

    def top4(c):
        cols = []
        for k in range(_TOPK):
            v = jnp.min(c, axis=1, keepdims=True)
            idx = jnp.min(jnp.where(c <= v, col, _BIG), axis=1, keepdims=True)
            cols.append(idx)
            if k < _TOPK - 1:
                c = jnp.where(col == idx, _BIG, c)
        return cols

    cost_p = (jnp.abs(pcx - tcx) + jnp.abs(pcy - tcy)
              + jnp.abs(pw - tw) + jnp.abs(ph - th))
    acx = anc_t_ref[0:1, :]
    acy = anc_t_ref[1:2, :]
    aw = anc_t_ref[2:3, :]
    ah = anc_t_ref[3:4, :]
    cost_a = (jnp.abs(acx - tcx) + jnp.abs(acy - tcy)
              + jnp.abs(aw - tw) + jnp.abs(ah - th))
    cp = top4(cost_p)
    ca = top4(cost_a)
    out_ref[0] = jnp.concatenate(cp + ca, axis=1).astype(jnp.int32)

    src_c = jnp.concatenate(
        [cols[k] for k in range(_TOPK) for cols in (cp, ca)], axis=0)
    src_r = jnp.transpose(src_c)
    j_col = lax.broadcasted_iota(jnp.int32, (j_n, 1), 0)
    j_row = lax.broadcasted_iota(jnp.int32, (1, j_n), 1)
    later = jnp.max(
        jnp.where(jnp.logical_and(src_c == src_r, j_row > j_col), 1.0, 0.0),
        axis=1, keepdims=True)
    last_ref[0] = jnp.transpose(1.0 - later)


def _loss_kernel(pc_ref, pbn_ref, pbt_ref, anct_ref, tgt_t_ref, tgtjt_ref,
                 labj_ref, srcr_ref, last_ref, mskf_ref, out_ref, acc_ref):
    b = pl.program_id(0)
    t = pl.program_id(1)
    nb = pl.num_programs(0)
    nt = pl.num_programs(1)
    tile = pbn_ref.shape[1]
    c_dim = pc_ref.shape[2]
    j_n = srcr_ref.shape[2]

    @pl.when(jnp.logical_and(b == 0, t == 0))
    def _():
        acc_ref[0] = 0.0
        acc_ref[1] = 0.0
        acc_ref[2] = 0.0

    base_i = t * tile
    a_col_i = lax.broadcasted_iota(jnp.int32, (tile, 1), 0) + base_i
    src_ri = srcr_ref[0]
    mask_bf = jnp.where(a_col_i == src_ri, 1.0, 0.0).astype(jnp.bfloat16)

    in_tile = jnp.logical_and(src_ri >= base_i,
                              src_ri < base_i + tile).astype(jnp.float32)

    anct = anct_ref[...]
    pbt = pbt_ref[0]
    anct_hi = anct.astype(jnp.bfloat16)
    pbt_hi = pbt.astype(jnp.bfloat16)
    anct_lo = (anct - anct_hi.astype(jnp.float32)).astype(jnp.bfloat16)
    pbt_lo = (pbt - pbt_hi.astype(jnp.float32)).astype(jnp.bfloat16)
    vals16 = jnp.concatenate([anct_hi, pbt_hi, anct_lo, pbt_lo], axis=0)
    gath16 = lax.dot_general(vals16, mask_bf, (((1,), (0,)), ((), ())),
                             preferred_element_type=jnp.float32)
    anc8 = gath16[0:4, :] + gath16[8:12, :]
    pb8 = gath16[4:8, :] + gath16[12:16, :]
    acx = anc8[0:1, :]
    acy = anc8[1:2, :]
    aw = anc8[2:3, :]
    ah = anc8[3:4, :]
    ax0 = acx - 0.5 * aw
    ay0 = acy - 0.5 * ah
    ax1 = acx + 0.5 * aw
    ay1 = acy + 0.5 * ah
    px0 = pb8[0:1, :]
    py0 = pb8[1:2, :]
    px1 = pb8[2:3, :]
    py1 = pb8[3:4, :]

    tjt = tgtjt_ref[0]
    tx0 = tjt[0:1, :]
    ty0 = tjt[1:2, :]
    tx1 = tjt[2:3, :]
    ty1 = tjt[3:4, :]
    area_t = (tx1 - tx0) * (ty1 - ty0)

    area_a = (ax1 - ax0) * (ay1 - ay0)
    iw = jnp.maximum(jnp.minimum(ax1, tx1) - jnp.maximum(ax0, tx0), 0.0)
    ih = jnp.maximum(jnp.minimum(ay1, ty1) - jnp.maximum(ay0, ty0), 0.0)
    inter = iw * ih
    union = area_a + area_t - inter
    pos_iou = inter / jnp.maximum(union, 1e-8)

    keep = (pos_iou >= _IOU_THRESH).astype(jnp.float32)
    tgt_cls_o = jnp.where(pos_iou < _IOU_THRESH, -1.0, labj_ref[0])
    is_last = last_ref[0]

    enc_v = tgt_cls_o * is_last
    enc2 = jnp.concatenate(
        [jnp.transpose(enc_v), jnp.transpose(is_last)],
        axis=1).astype(jnp.bfloat16)
    agg = lax.dot_general(mask_bf, enc2, (((1,), (0,)), ((), ())),
                          preferred_element_type=jnp.float32)
    enc_sum = agg[:, 0:1]
    matched = agg[:, 1:2] > 0.5

    pb = pbn_ref[0]
    qx0 = pb[:, 0:1]
    qy0 = pb[:, 1:2]
    qx1 = pb[:, 2:3]
    qy1 = pb[:, 3:4]
    gx0 = tgt_t_ref[0, 0:1, :]
    gy0 = tgt_t_ref[0, 1:2, :]
    gx1 = tgt_t_ref[0, 2:3, :]
    gy1 = tgt_t_ref[0, 3:4, :]
    area_q = (qx1 - qx0) * (qy1 - qy0)
    area_g = (gx1 - gx0) * (gy1 - gy0)
    iw2 = jnp.maximum(jnp.minimum(qx1, gx1) - jnp.maximum(qx0, gx0), 0.0)
    ih2 = jnp.maximum(jnp.minimum(qy1, gy1) - jnp.maximum(qy0, gy0), 0.0)
    inter2 = iw2 * ih2
    union2 = area_q + area_g - inter2
    marg = inter2 - _IGNORE_THRESH * jnp.maximum(union2, 1e-8)
    ignore = jnp.max(marg, axis=1, keepdims=True) > 0.0

    gt_cls = jnp.where(matched, enc_sum,
                       jnp.where(ignore, -1.0, float(_NUM_CLASSES)))
    valid_f = (gt_cls >= 0.0).astype(jnp.float32) * (1.0 - mskf_ref[0])
    fg_f = jnp.logical_and(gt_cls >= 0.0,
                           gt_cls < _NUM_CLASSES - 0.5).astype(jnp.float32)
    fg_cls = gt_cls * fg_f

    cls_iota = lax.broadcasted_iota(jnp.int32, (tile, c_dim),
                                    1).astype(jnp.float32)
    x = pc_ref[0]
    p = jax.nn.sigmoid(x)
    sp = jnp.maximum(x, 0.0) + jnp.log1p(jnp.exp(-jnp.abs(x)))
    f0 = (1.0 - _ALPHA) * sp * (p * p)
    bg_row = jnp.sum(f0, axis=1, keepdims=True)
    xc = jnp.sum(jnp.where(cls_iota == fg_cls, x, 0.0), axis=1,
                 keepdims=True)
    pc = jax.nn.sigmoid(xc)
    spc = jnp.maximum(xc, 0.0) + jnp.log1p(jnp.exp(-jnp.abs(xc)))
    f0c = (1.0 - _ALPHA) * spc * (pc * pc)
    omp = 1.0 - pc
    f1c = _ALPHA * (spc - xc) * (omp * omp)
    cls_part = jnp.sum(valid_f * (bg_row + fg_f * (f1c - f0c)))

    area_p2 = (px1 - px0) * (py1 - py0)
    iw3 = jnp.maximum(jnp.minimum(px1, tx1) - jnp.maximum(px0, tx0), 0.0)
    ih3 = jnp.maximum(jnp.minimum(py1, ty1) - jnp.maximum(py0, ty0), 0.0)
    inter3 = iw3 * ih3
    union3 = area_p2 + area_t - inter3
    iou3 = inter3 / jnp.maximum(union3, 1e-8)
    ew = jnp.maximum(jnp.maximum(px1, tx1) - jnp.minimum(px0, tx0), 0.0)
    eh = jnp.maximum(jnp.maximum(py1, ty1) - jnp.minimum(py0, ty0), 0.0)
    area_e = ew * eh
    gi = iou3 - (area_e - union3) / jnp.maximum(area_e, 1e-8)
    reg_part = jnp.sum(in_tile * keep * (1.0 - gi))

    acc_ref[0] = acc_ref[0] + cls_part
    acc_ref[1] = acc_ref[1] + reg_part
    acc_ref[2] = acc_ref[2] + jnp.sum(fg_f)

    @pl.when(jnp.logical_and(b == nb - 1, t == nt - 1))
    def _():
        num_fg = jnp.maximum(acc_ref[2], 1.0)
        lc = acc_ref[0] / num_fg
        lr = acc_ref[1] / num_fg
        out_ref[0] = lc
        out_ref[1] = lr
        out_ref[2] = lc + lr


def kernel(pred_cls, pred_box, anchors, mask, tgt_boxes, tgt_labels):
    bn, m, c = pred_cls.shape
    g = tgt_boxes.shape[1]
    j_n = 2 * _TOPK * g

    pred_box_t = pred_box.transpose(0, 2, 1)
    anchors_t = anchors.T
    tgt_t = tgt_boxes.transpose(0, 2, 1)

    matches, is_last = pl.pallas_call(
        _match_kernel,
        grid=(bn,),
        in_specs=[
            pl.BlockSpec((1, 4, m), lambda b: (b, 0, 0)),
            pl.BlockSpec((4, m), lambda b: (0, 0)),
            pl.BlockSpec((1, g, 4), lambda b: (b, 0, 0)),
        ],
        out_specs=[
            pl.BlockSpec((1, g, 2 * _TOPK), lambda b: (b, 0, 0)),
            pl.BlockSpec((1, 1, j_n), lambda b: (b, 0, 0)),
        ],
        out_shape=[
            jax.ShapeDtypeStruct((bn, g, 2 * _TOPK), jnp.int32),
            jax.ShapeDtypeStruct((bn, 1, j_n), jnp.float32),
        ],
    )(pred_box_t, anchors_t, tgt_boxes)

    m_t = matches.transpose(0, 2, 1)
    idx_p = m_t[:, :_TOPK, :]
    idx_a = m_t[:, _TOPK:, :]
    src_row = jnp.stack([idx_p, idx_a], axis=2).reshape(bn, 1, j_n)
    tgt_jt = jnp.tile(tgt_t, (1, 1, 2 * _TOPK))
    lab_j = jnp.tile(tgt_labels.astype(jnp.float32),
                     (1, 2 * _TOPK)).reshape(bn, 1, j_n)
    mask_f = mask.astype(jnp.float32).reshape(bn, m, 1)

    tile = 2048
    t_n = m // tile
    out = pl.pallas_call(
        _loss_kernel,
        grid=(bn, t_n),
        in_specs=[
            pl.BlockSpec((1, tile, c), lambda b, t: (b, t, 0)),
            pl.BlockSpec((1, tile, 4), lambda b, t: (b, t, 0)),
            pl.BlockSpec((1, 4, tile), lambda b, t: (b, 0, t)),
            pl.BlockSpec((4, tile), lambda b, t: (0, t)),
            pl.BlockSpec((1, 4, g), lambda b, t: (b, 0, 0)),
            pl.BlockSpec((1, 4, j_n), lambda b, t: (b, 0, 0)),
            pl.BlockSpec((1, 1, j_n), lambda b, t: (b, 0, 0)),
            pl.BlockSpec((1, 1, j_n), lambda b, t: (b, 0, 0)),
            pl.BlockSpec((1, 1, j_n), lambda b, t: (b, 0, 0)),
            pl.BlockSpec((1, tile, 1), lambda b, t: (b, t, 0)),
        ],
        out_specs=pl.BlockSpec(memory_space=pltpu.SMEM),
        out_shape=jax.ShapeDtypeStruct((3,), jnp.float32),
        scratch_shapes=[pltpu.SMEM((3,), jnp.float32)],
    )(pred_cls, pred_box, pred_box_t, anchors_t, tgt_t, tgt_jt, lab_j,
      src_row, is_last, mask_f)

    return out[0], out[1], out[2]


# speedup vs baseline: 2.1190x; 1.0162x over previous
import jax
import jax.numpy as jnp
from jax import lax
from jax.experimental import pallas as pl
from jax.experimental.pallas import tpu as pltpu

_NUM_CLASSES = 80
_ALPHA, _GAMMA = 0.25, 2.0
_TOPK = 4
_IGNORE_THRESH, _IOU_THRESH = 0.7, 0.15
_BIG = 3.0e38
_CHUNK = 2048


def _match_kernel(pred_t_ref, anc_t_ref, tgtn_ref, tgtjt_ref, labj_ref,
                  agg_ref, reg_ref):
    b = pl.program_id(0)
    m = pred_t_ref.shape[2]
    g = tgtn_ref.shape[1]
    j_n = 2 * _TOPK * g
    x0 = pred_t_ref[0, 0:1, :]
    y0 = pred_t_ref[0, 1:2, :]
    x1 = pred_t_ref[0, 2:3, :]
    y1 = pred_t_ref[0, 3:4, :]
    pcx = (x0 + x1) * 0.5
    pcy = (y0 + y1) * 0.5
    pw = x1 - x0
    ph = y1 - y0

    tb = tgtn_ref[0]
    tcx = (tb[:, 0:1] + tb[:, 2:3]) * 0.5
    tcy = (tb[:, 1:2] + tb[:, 3:4]) * 0.5
    tw = tb[:, 2:3] - tb[:, 0:1]
    th = tb[:, 3:4] - tb[:, 1:2]

    col = lax.broadcasted_iota(jnp.int32, (1, m), 1).astype(jnp.float32)

    def top4(c):
        cols = []
        for k in range(_TOPK):
            v = jnp.min(c, axis=1, keepdims=True)
            idx = jnp.min(jnp.where(c <= v, col, _BIG), axis=1, keepdims=True)
            cols.append(idx)
            if k < _TOPK - 1:
                c = jnp.where(col == idx, _BIG, c)
        return cols

    cost_p = (jnp.abs(pcx - tcx) + jnp.abs(pcy - tcy)
              + jnp.abs(pw - tw) + jnp.abs(ph - th))
    acx_r = anc_t_ref[0:1, :]
    acy_r = anc_t_ref[1:2, :]
    aw_r = anc_t_ref[2:3, :]
    ah_r = anc_t_ref[3:4, :]
    cost_a = (jnp.abs(acx_r - tcx) + jnp.abs(acy_r - tcy)
              + jnp.abs(aw_r - tw) + jnp.abs(ah_r - th))
    cp = top4(cost_p)
    ca = top4(cost_a)

    src_c = jnp.concatenate(
        [cols[k] for k in range(_TOPK) for cols in (cp, ca)], axis=0)
    src_r = jnp.transpose(src_c)
    j_col = lax.broadcasted_iota(jnp.int32, (j_n, 1), 0)
    j_row = lax.broadcasted_iota(jnp.int32, (1, j_n), 1)
    later = jnp.max(
        jnp.where(jnp.logical_and(src_c == src_r, j_row > j_col), 1.0, 0.0),
        axis=1, keepdims=True)
    is_last = jnp.transpose(1.0 - later)

    anct = anc_t_ref[...]
    pbt = pred_t_ref[0]
    anct_hi = anct.astype(jnp.bfloat16)
    pbt_hi = pbt.astype(jnp.bfloat16)
    anct_lo = (anct - anct_hi.astype(jnp.float32)).astype(jnp.bfloat16)
    pbt_lo = (pbt - pbt_hi.astype(jnp.float32)).astype(jnp.bfloat16)
    vals16 = jnp.concatenate([anct_hi, pbt_hi, anct_lo, pbt_lo], axis=0)

    ch_iota = lax.broadcasted_iota(jnp.int32, (_CHUNK, 1), 0)
    gath16 = jnp.zeros((16, j_n), jnp.float32)
    masks = []
    for i in range(m // _CHUNK):
        mask_i = jnp.where(ch_iota + (i * _CHUNK) == src_r, 1.0,
                           0.0).astype(jnp.bfloat16)
        masks.append(mask_i)
        gath16 = gath16 + lax.dot_general(
            vals16[:, i * _CHUNK:(i + 1) * _CHUNK], mask_i,
            (((1,), (0,)), ((), ())), preferred_element_type=jnp.float32)

    anc8 = gath16[0:4, :] + gath16[8:12, :]
    pb8 = gath16[4:8, :] + gath16[12:16, :]
    acx = anc8[0:1, :]
    acy = anc8[1:2, :]
    aw = anc8[2:3, :]
    ah = anc8[3:4, :]
    ax0 = acx - 0.5 * aw
    ay0 = acy - 0.5 * ah
    ax1 = acx + 0.5 * aw
    ay1 = acy + 0.5 * ah
    px0 = pb8[0:1, :]
    py0 = pb8[1:2, :]
    px1 = pb8[2:3, :]
    py1 = pb8[3:4, :]

    tjt = tgtjt_ref[0]
    tx0 = tjt[0:1, :]
    ty0 = tjt[1:2, :]
    tx1 = tjt[2:3, :]
    ty1 = tjt[3:4, :]
    area_t = (tx1 - tx0) * (ty1 - ty0)

    area_a = (ax1 - ax0) * (ay1 - ay0)
    iw = jnp.maximum(jnp.minimum(ax1, tx1) - jnp.maximum(ax0, tx0), 0.0)
    ih = jnp.maximum(jnp.minimum(ay1, ty1) - jnp.maximum(ay0, ty0), 0.0)
    inter = iw * ih
    union = area_a + area_t - inter
    pos_iou = inter / jnp.maximum(union, 1e-8)

    keep = (pos_iou >= _IOU_THRESH).astype(jnp.float32)
    tgt_cls_o = jnp.where(pos_iou < _IOU_THRESH, -1.0, labj_ref[0])

    enc_v = tgt_cls_o * is_last
    enc2 = jnp.concatenate(
        [jnp.transpose(enc_v), jnp.transpose(is_last)],
        axis=1).astype(jnp.bfloat16)
    for i in range(m // _CHUNK):
        agg_ref[0, i * _CHUNK:(i + 1) * _CHUNK, :] = lax.dot_general(
            masks[i], enc2, (((1,), (0,)), ((), ())),
            preferred_element_type=jnp.float32)

    area_p2 = (px1 - px0) * (py1 - py0)
    iw3 = jnp.maximum(jnp.minimum(px1, tx1) - jnp.maximum(px0, tx0), 0.0)
    ih3 = jnp.maximum(jnp.minimum(py1, ty1) - jnp.maximum(py0, ty0), 0.0)
    inter3 = iw3 * ih3
    union3 = area_p2 + area_t - inter3
    iou3 = inter3 / jnp.maximum(union3, 1e-8)
    ew = jnp.maximum(jnp.maximum(px1, tx1) - jnp.minimum(px0, tx0), 0.0)
    eh = jnp.maximum(jnp.maximum(py1, ty1) - jnp.minimum(py0, ty0), 0.0)
    area_e = ew * eh
    gi = iou3 - (area_e - union3) / jnp.maximum(area_e, 1e-8)
    reg_ref[b] = jnp.sum(keep * (1.0 - gi))


def _loss_kernel(pc_ref, pbn_ref, tgt_t_ref, agg_ref, mskf_ref, reg_sm_ref,
                 out_ref, acc_ref):
    b = pl.program_id(0)
    t = pl.program_id(1)
    nb = pl.num_programs(0)
    nt = pl.num_programs(1)
    tile = pbn_ref.shape[1]
    c_dim = pc_ref.shape[2]

    @pl.when(jnp.logical_and(b == 0, t == 0))
    def _():
        acc_ref[0] = 0.0
        acc_ref[1] = 0.0
        acc_ref[2] = 0.0

    @pl.when(t == 0)
    def _():
        acc_ref[1] = acc_ref[1] + reg_sm_ref[b]

    agg = agg_ref[0]
    enc_sum = agg[:, 0:1]
    matched = agg[:, 1:2] > 0.5

    pb = pbn_ref[0]
    qx0 = pb[:, 0:1]
    qy0 = pb[:, 1:2]
    qx1 = pb[:, 2:3]
    qy1 = pb[:, 3:4]
    gx0 = tgt_t_ref[0, 0:1, :]
    gy0 = tgt_t_ref[0, 1:2, :]
    gx1 = tgt_t_ref[0, 2:3, :]
    gy1 = tgt_t_ref[0, 3:4, :]
    area_q = (qx1 - qx0) * (qy1 - qy0)
    area_g = (gx1 - gx0) * (gy1 - gy0)
    iw2 = jnp.maximum(jnp.minimum(qx1, gx1) - jnp.maximum(qx0, gx0), 0.0)
    ih2 = jnp.maximum(jnp.minimum(qy1, gy1) - jnp.maximum(qy0, gy0), 0.0)
    inter2 = iw2 * ih2
    union2 = area_q + area_g - inter2
    marg = inter2 - _IGNORE_THRESH * jnp.maximum(union2, 1e-8)
    ignore = jnp.max(marg, axis=1, keepdims=True) > 0.0

    gt_cls = jnp.where(matched, enc_sum,
                       jnp.where(ignore, -1.0, float(_NUM_CLASSES)))
    valid_f = (gt_cls >= 0.0).astype(jnp.float32) * (1.0 - mskf_ref[0])
    fg_f = jnp.logical_and(gt_cls >= 0.0,
                           gt_cls < _NUM_CLASSES - 0.5).astype(jnp.float32)
    fg_cls = gt_cls * fg_f

    cls_iota = lax.broadcasted_iota(jnp.int32, (tile, c_dim),
                                    1).astype(jnp.float32)
    t_mat = (cls_iota == fg_cls).astype(jnp.float32) * fg_f
    x = pc_ref[0]
    p = jax.nn.sigmoid(x)
    ce = jnp.maximum(x, 0.0) - x * t_mat + jnp.log1p(jnp.exp(-jnp.abs(x)))
    p_t = p * t_mat + (1.0 - p) * (1.0 - t_mat)
    one_m = 1.0 - p_t
    focal = ce * (one_m * one_m)
    alpha_t = _ALPHA * t_mat + (1.0 - _ALPHA) * (1.0 - t_mat)
    cls_part = jnp.sum(alpha_t * focal * valid_f)

    acc_ref[0] = acc_ref[0] + cls_part
    acc_ref[2] = acc_ref[2] + jnp.sum(fg_f)

    @pl.when(jnp.logical_and(b == nb - 1, t == nt - 1))
    def _():
        num_fg = jnp.maximum(acc_ref[2], 1.0)
        lc = acc_ref[0] / num_fg
        lr = acc_ref[1] / num_fg
        out_ref[0] = lc
        out_ref[1] = lr
        out_ref[2] = lc + lr


def kernel(pred_cls, pred_box, anchors, mask, tgt_boxes, tgt_labels):
    bn, m, c = pred_cls.shape
    g = tgt_boxes.shape[1]
    j_n = 2 * _TOPK * g

    pred_box_t = pred_box.transpose(0, 2, 1)
    anchors_t = anchors.T
    tgt_t = tgt_boxes.transpose(0, 2, 1)
    tgt_jt = jnp.tile(tgt_t, (1, 1, 2 * _TOPK))
    lab_j = jnp.tile(tgt_labels.astype(jnp.float32),
                     (1, 2 * _TOPK)).reshape(bn, 1, j_n)
    mask_f = mask.astype(jnp.float32).reshape(bn, m, 1)

    agg, reg_b = pl.pallas_call(
        _match_kernel,
        grid=(bn,),
        in_specs=[
            pl.BlockSpec((1, 4, m), lambda b: (b, 0, 0)),
            pl.BlockSpec((4, m), lambda b: (0, 0)),
            pl.BlockSpec((1, g, 4), lambda b: (b, 0, 0)),
            pl.BlockSpec((1, 4, j_n), lambda b: (b, 0, 0)),
            pl.BlockSpec((1, 1, j_n), lambda b: (b, 0, 0)),
        ],
        out_specs=[
            pl.BlockSpec((1, m, 2), lambda b: (b, 0, 0)),
            pl.BlockSpec(memory_space=pltpu.SMEM),
        ],
        out_shape=[
            jax.ShapeDtypeStruct((bn, m, 2), jnp.float32),
            jax.ShapeDtypeStruct((bn,), jnp.float32),
        ],
    )(pred_box_t, anchors_t, tgt_boxes, tgt_jt, lab_j)

    tile = 4096
    t_n = m // tile
    out = pl.pallas_call(
        _loss_kernel,
        grid=(bn, t_n),
        in_specs=[
            pl.BlockSpec((1, tile, c), lambda b, t: (b, t, 0)),
            pl.BlockSpec((1, tile, 4), lambda b, t: (b, t, 0)),
            pl.BlockSpec((1, 4, g), lambda b, t: (b, 0, 0)),
            pl.BlockSpec((1, tile, 2), lambda b, t: (b, t, 0)),
            pl.BlockSpec((1, tile, 1), lambda b, t: (b, t, 0)),
            pl.BlockSpec(memory_space=pltpu.SMEM),
        ],
        out_specs=pl.BlockSpec(memory_space=pltpu.SMEM),
        out_shape=jax.ShapeDtypeStruct((3,), jnp.float32),
        scratch_shapes=[pltpu.SMEM((3,), jnp.float32)],
    )(pred_cls, pred_box, tgt_t, agg, mask_f, reg_b)

    return out[0], out[1], out[2]


# speedup vs baseline: 2.2873x; 1.0795x over previous
import jax
import jax.numpy as jnp
from jax import lax
from jax.experimental import pallas as pl
from jax.experimental.pallas import tpu as pltpu

_NUM_CLASSES = 80
_ALPHA, _GAMMA = 0.25, 2.0
_TOPK = 4
_IGNORE_THRESH, _IOU_THRESH = 0.7, 0.15
_BIG = 3.0e38


def _match_kernel(pred_t_ref, anc_t_ref, tgtn_ref, out_ref, last_ref):
    m = pred_t_ref.shape[2]
    g = tgtn_ref.shape[1]
    j_n = 2 * _TOPK * g
    x0 = pred_t_ref[0, 0:1, :]
    y0 = pred_t_ref[0, 1:2, :]
    x1 = pred_t_ref[0, 2:3, :]
    y1 = pred_t_ref[0, 3:4, :]
    pcx = (x0 + x1) * 0.5
    pcy = (y0 + y1) * 0.5
    pw = x1 - x0
    ph = y1 - y0

    tb = tgtn_ref[0]
    tx0 = tb[:, 0:1]
    ty0 = tb[:, 1:2]
    tx1 = tb[:, 2:3]
    ty1 = tb[:, 3:4]
    tcx = (tx0 + tx1) * 0.5
    tcy = (ty0 + ty1) * 0.5
    tw = tx1 - tx0
    th = ty1 - ty0

    col = lax.broadcasted_iota(jnp.int32, (1, m), 1).astype(jnp.float32)

    def top4(c):
        cols = []
        for k in range(_TOPK):
            v = jnp.min(c, axis=1, keepdims=True)
            idx = jnp.min(jnp.where(c <= v, col, _BIG), axis=1, keepdims=True)
            cols.append(idx)
            if k < _TOPK - 1:
                c = jnp.where(col == idx, _BIG, c)
        return cols

    cost_p = (jnp.abs(pcx - tcx) + jnp.abs(pcy - tcy)
              + jnp.abs(pw - tw) + jnp.abs(ph - th))
    acx = anc_t_ref[0:1, :]
    acy = anc_t_ref[1:2, :]
    aw = anc_t_ref[2:3, :]
    ah = anc_t_ref[3:4, :]
    cost_a = (jnp.abs(acx - tcx) + jnp.abs(acy - tcy)
              + jnp.abs(aw - tw) + jnp.abs(ah - th))
    cp = top4(cost_p)
    ca = top4(cost_a)
    out_ref[0] = jnp.concatenate(cp + ca, axis=1).astype(jnp.int32)

    src_c = jnp.concatenate(
        [cols[k] for k in range(_TOPK) for cols in (cp, ca)], axis=0)
    src_r = jnp.transpose(src_c)
    j_col = lax.broadcasted_iota(jnp.int32, (j_n, 1), 0)
    j_row = lax.broadcasted_iota(jnp.int32, (1, j_n), 1)
    later = jnp.max(
        jnp.where(jnp.logical_and(src_c == src_r, j_row > j_col), 1.0, 0.0),
        axis=1, keepdims=True)
    last_ref[0] = jnp.transpose(1.0 - later)


def _loss_kernel(pc_ref, pbn_ref, pbt_ref, anct_ref, tgt_t_ref, tgtjt_ref,
                 labj_ref, srcr_ref, last_ref, mskf_ref, out_ref, acc_ref):
    b = pl.program_id(0)
    t = pl.program_id(1)
    nb = pl.num_programs(0)
    nt = pl.num_programs(1)
    tile = pbn_ref.shape[1]
    c_dim = pc_ref.shape[2]
    j_n = srcr_ref.shape[2]

    @pl.when(jnp.logical_and(b == 0, t == 0))
    def _():
        acc_ref[0] = 0.0
        acc_ref[1] = 0.0
        acc_ref[2] = 0.0

    base_i = t * tile
    a_col_i = lax.broadcasted_iota(jnp.int32, (tile, 1), 0) + base_i
    src_ri = srcr_ref[0]
    mask_bf = jnp.where(a_col_i == src_ri, 1.0, 0.0).astype(jnp.bfloat16)

    in_tile = jnp.logical_and(src_ri >= base_i,
                              src_ri < base_i + tile).astype(jnp.float32)

    anct = anct_ref[...]
    pbt = pbt_ref[0]
    anct_hi = anct.astype(jnp.bfloat16)
    pbt_hi = pbt.astype(jnp.bfloat16)
    anct_lo = (anct - anct_hi.astype(jnp.float32)).astype(jnp.bfloat16)
    pbt_lo = (pbt - pbt_hi.astype(jnp.float32)).astype(jnp.bfloat16)
    vals16 = jnp.concatenate([anct_hi, pbt_hi, anct_lo, pbt_lo], axis=0)
    gath16 = lax.dot_general(vals16, mask_bf, (((1,), (0,)), ((), ())),
                             preferred_element_type=jnp.float32)
    anc8 = gath16[0:4, :] + gath16[8:12, :]
    pb8 = gath16[4:8, :] + gath16[12:16, :]
    acx = anc8[0:1, :]
    acy = anc8[1:2, :]
    aw = anc8[2:3, :]
    ah = anc8[3:4, :]
    ax0 = acx - 0.5 * aw
    ay0 = acy - 0.5 * ah
    ax1 = acx + 0.5 * aw
    ay1 = acy + 0.5 * ah
    px0 = pb8[0:1, :]
    py0 = pb8[1:2, :]
    px1 = pb8[2:3, :]
    py1 = pb8[3:4, :]

    tjt = tgtjt_ref[0]
    tx0 = tjt[0:1, :]
    ty0 = tjt[1:2, :]
    tx1 = tjt[2:3, :]
    ty1 = tjt[3:4, :]
    area_t = (tx1 - tx0) * (ty1 - ty0)

    area_a = (ax1 - ax0) * (ay1 - ay0)
    iw = jnp.maximum(jnp.minimum(ax1, tx1) - jnp.maximum(ax0, tx0), 0.0)
    ih = jnp.maximum(jnp.minimum(ay1, ty1) - jnp.maximum(ay0, ty0), 0.0)
    inter = iw * ih
    union = area_a + area_t - inter
    pos_iou = inter / jnp.maximum(union, 1e-8)

    keep = (pos_iou >= _IOU_THRESH).astype(jnp.float32)
    tgt_cls_o = jnp.where(pos_iou < _IOU_THRESH, -1.0, labj_ref[0])
    is_last = last_ref[0]

    enc_v = tgt_cls_o * is_last
    enc2 = jnp.concatenate(
        [jnp.transpose(enc_v), jnp.transpose(is_last)],
        axis=1).astype(jnp.bfloat16)
    agg = lax.dot_general(mask_bf, enc2, (((1,), (0,)), ((), ())),
                          preferred_element_type=jnp.float32)
    enc_sum = agg[:, 0:1]
    matched = agg[:, 1:2] > 0.5

    pb = pbn_ref[0]
    qx0 = pb[:, 0:1]
    qy0 = pb[:, 1:2]
    qx1 = pb[:, 2:3]
    qy1 = pb[:, 3:4]
    gx0 = tgt_t_ref[0, 0:1, :]
    gy0 = tgt_t_ref[0, 1:2, :]
    gx1 = tgt_t_ref[0, 2:3, :]
    gy1 = tgt_t_ref[0, 3:4, :]
    area_q = (qx1 - qx0) * (qy1 - qy0)
    area_g = (gx1 - gx0) * (gy1 - gy0)
    iw2 = jnp.maximum(jnp.minimum(qx1, gx1) - jnp.maximum(qx0, gx0), 0.0)
    ih2 = jnp.maximum(jnp.minimum(qy1, gy1) - jnp.maximum(qy0, gy0), 0.0)
    inter2 = iw2 * ih2
    union2 = area_q + area_g - inter2
    marg = inter2 - _IGNORE_THRESH * jnp.maximum(union2, 1e-8)
    ignore = jnp.max(marg, axis=1, keepdims=True) > 0.0

    gt_cls = jnp.where(matched, enc_sum,
                       jnp.where(ignore, -1.0, float(_NUM_CLASSES)))
    valid_f = (gt_cls >= 0.0).astype(jnp.float32) * (1.0 - mskf_ref[0])
    fg_f = jnp.logical_and(gt_cls >= 0.0,
                           gt_cls < _NUM_CLASSES - 0.5).astype(jnp.float32)
    fg_cls = gt_cls * fg_f

    cls_iota = lax.broadcasted_iota(jnp.int32, (tile, c_dim),
                                    1).astype(jnp.float32)
    t_mat = (cls_iota == fg_cls).astype(jnp.float32) * fg_f
    x = pc_ref[0]
    p = jax.nn.sigmoid(x)
    ce = jnp.maximum(x, 0.0) - x * t_mat + jnp.log1p(jnp.exp(-jnp.abs(x)))
    p_t = p * t_mat + (1.0 - p) * (1.0 - t_mat)
    one_m = 1.0 - p_t
    focal = ce * (one_m * one_m)
    alpha_t = _ALPHA * t_mat + (1.0 - _ALPHA) * (1.0 - t_mat)
    cls_part = jnp.sum(alpha_t * focal * valid_f)

    area_p2 = (px1 - px0) * (py1 - py0)
    iw3 = jnp.maximum(jnp.minimum(px1, tx1) - jnp.maximum(px0, tx0), 0.0)
    ih3 = jnp.maximum(jnp.minimum(py1, ty1) - jnp.maximum(py0, ty0), 0.0)
    inter3 = iw3 * ih3
    union3 = area_p2 + area_t - inter3
    iou3 = inter3 / jnp.maximum(union3, 1e-8)
    ew = jnp.maximum(jnp.maximum(px1, tx1) - jnp.minimum(px0, tx0), 0.0)
    eh = jnp.maximum(jnp.maximum(py1, ty1) - jnp.minimum(py0, ty0), 0.0)
    area_e = ew * eh
    gi = iou3 - (area_e - union3) / jnp.maximum(area_e, 1e-8)
    reg_part = jnp.sum(in_tile * keep * (1.0 - gi))

    acc_ref[0] = acc_ref[0] + cls_part
    acc_ref[1] = acc_ref[1] + reg_part
    acc_ref[2] = acc_ref[2] + jnp.sum(fg_f)

    @pl.when(jnp.logical_and(b == nb - 1, t == nt - 1))
    def _():
        num_fg = jnp.maximum(acc_ref[2], 1.0)
        lc = acc_ref[0] / num_fg
        lr = acc_ref[1] / num_fg
        out_ref[0] = lc
        out_ref[1] = lr
        out_ref[2] = lc + lr


def kernel(pred_cls, pred_box, anchors, mask, tgt_boxes, tgt_labels):
    bn, m, c = pred_cls.shape
    g = tgt_boxes.shape[1]
    j_n = 2 * _TOPK * g

    pred_box_t = pred_box.transpose(0, 2, 1)
    anchors_t = anchors.T
    tgt_t = tgt_boxes.transpose(0, 2, 1)

    matches, is_last = pl.pallas_call(
        _match_kernel,
        grid=(bn,),
        in_specs=[
            pl.BlockSpec((1, 4, m), lambda b: (b, 0, 0)),
            pl.BlockSpec((4, m), lambda b: (0, 0)),
            pl.BlockSpec((1, g, 4), lambda b: (b, 0, 0)),
        ],
        out_specs=[
            pl.BlockSpec((1, g, 2 * _TOPK), lambda b: (b, 0, 0)),
            pl.BlockSpec((1, 1, j_n), lambda b: (b, 0, 0)),
        ],
        out_shape=[
            jax.ShapeDtypeStruct((bn, g, 2 * _TOPK), jnp.int32),
            jax.ShapeDtypeStruct((bn, 1, j_n), jnp.float32),
        ],
    )(pred_box_t, anchors_t, tgt_boxes)

    m_t = matches.transpose(0, 2, 1)
    idx_p = m_t[:, :_TOPK, :]
    idx_a = m_t[:, _TOPK:, :]
    src_row = jnp.stack([idx_p, idx_a], axis=2).reshape(bn, 1, j_n)
    tgt_jt = jnp.tile(tgt_t, (1, 1, 2 * _TOPK))
    lab_j = jnp.tile(tgt_labels.astype(jnp.float32),
                     (1, 2 * _TOPK)).reshape(bn, 1, j_n)
    mask_f = mask.astype(jnp.float32).reshape(bn, m, 1)

    tile = 8192
    t_n = m // tile
    out = pl.pallas_call(
        _loss_kernel,
        grid=(bn, t_n),
        in_specs=[
            pl.BlockSpec((1, tile, c), lambda b, t: (b, t, 0)),
            pl.BlockSpec((1, tile, 4), lambda b, t: (b, t, 0)),
            pl.BlockSpec((1, 4, tile), lambda b, t: (b, 0, t)),
            pl.BlockSpec((4, tile), lambda b, t: (0, t)),
            pl.BlockSpec((1, 4, g), lambda b, t: (b, 0, 0)),
            pl.BlockSpec((1, 4, j_n), lambda b, t: (b, 0, 0)),
            pl.BlockSpec((1, 1, j_n), lambda b, t: (b, 0, 0)),
            pl.BlockSpec((1, 1, j_n), lambda b, t: (b, 0, 0)),
            pl.BlockSpec((1, 1, j_n), lambda b, t: (b, 0, 0)),
            pl.BlockSpec((1, tile, 1), lambda b, t: (b, t, 0)),
        ],
        out_specs=pl.BlockSpec(memory_space=pltpu.SMEM),
        out_shape=jax.ShapeDtypeStruct((3,), jnp.float32),
        scratch_shapes=[pltpu.SMEM((3,), jnp.float32)],
    )(pred_cls, pred_box, pred_box_t, anchors_t, tgt_t, tgt_jt, lab_j,
      src_row, is_last, mask_f)

    return out[0], out[1], out[2]
